# Initial kernel scaffold; baseline (speedup 1.0000x reference)
import jax
import jax.numpy as jnp
from jax.experimental import pallas as pl


def kernel(x, edge_index, W1, b1, W2, b2, Wl, bl):
    raise NotImplementedError("write your pallas kernel here")


# speedup vs baseline: 79.0998x; 79.0998x over previous
import functools

import jax
import jax.numpy as jnp
from jax import lax
from jax.experimental import pallas as pl
from jax.experimental.pallas import tpu as pltpu
from jax.experimental.pallas import tpu_sc as plsc

N = 100000
E = 6400000
NC, NS, LANES = 2, 16, 16
NW = NC * NS
NP = 100352
NPW = NP // NW
NPS = NP // NS
CW = 8
K = 8
WIN = K * 128
EP = 6422528
NWIN = EP // WIN
WPT = NWIN // NW

_mesh = plsc.VectorSubcoreMesh(core_axis_name="c", subcore_axis_name="s")
_params = pltpu.CompilerParams(needs_layout_passes=False,
                               use_tc_tiling_on_sc=False)
_f32 = jnp.float32
_i32 = jnp.int32


def _iota():
    return lax.iota(_i32, LANES)


def _rsqrt_newton(d):
    i = lax.bitcast_convert_type(d, _i32)
    y = lax.bitcast_convert_type(jnp.int32(0x5F3759DF) - (i >> 1), _f32)
    for _ in range(3):
        y = y * (1.5 - 0.5 * d * y * y)
    return y


def _make_deg():
    @functools.partial(
        pl.kernel,
        out_type=jax.ShapeDtypeStruct((NC * NP,), _f32),
        mesh=_mesh,
        compiler_params=_params,
        scratch_types=[
            pltpu.VMEM((K, 128), _i32),
            pltpu.VMEM((128,), _f32),
            pltpu.VMEM((NPS,), _f32),
            pltpu.VMEM_SHARED((NP,), _f32),
            pltpu.SemaphoreType.DMA,
        ],
    )
    def deg_kernel(dst_hbm, zeros_hbm, out_hbm, dstbuf, ones_v, zbuf, deg_s,
                   sem):
        cid = lax.axis_index("c")
        sid = lax.axis_index("s")
        wid = sid * NC + cid
        pltpu.sync_copy(zeros_hbm.at[pl.ds(sid * NPS, NPS)], zbuf)
        pltpu.sync_copy(zbuf, deg_s.at[pl.ds(sid * NPS, NPS)])
        for i in range(128 // LANES):
            ones_v[pl.ds(i * LANES, LANES)] = jnp.full((LANES,), 1.0, _f32)
        plsc.subcore_barrier()

        def window(w, carry):
            pltpu.sync_copy(dst_hbm.at[w], dstbuf)
            handles = [
                pltpu.async_copy(ones_v, deg_s.at[dstbuf.at[j]], sem, add=True)
                for j in range(K)
            ]
            for h in handles:
                h.wait()
            return carry

        lax.fori_loop(wid * WPT, (wid + 1) * WPT, window, 0)
        plsc.subcore_barrier()
        pltpu.sync_copy(deg_s.at[pl.ds(sid * NPS, NPS)], zbuf)
        pltpu.sync_copy(zbuf, out_hbm.at[pl.ds(cid * NP + sid * NPS, NPS)])

    return deg_kernel


def _make_agg():
    @functools.partial(
        pl.kernel,
        out_type=jax.ShapeDtypeStruct((NC * NP, CW), _f32),
        mesh=_mesh,
        compiler_params=_params,
        scratch_types=[
            pltpu.VMEM((K, 128), _i32),
            pltpu.VMEM((K, 128), _i32),
            pltpu.VMEM((K, 128, CW), _f32),
            pltpu.VMEM((NPS, CW), _f32),
            pltpu.VMEM_SHARED((NP, CW), _f32),
            pltpu.SemaphoreType.DMA,
            pltpu.SemaphoreType.DMA,
        ],
    )
    def agg_kernel(g_hbm, src_hbm, dst_hbm, zeros_hbm, out_hbm,
                   srcbuf, dstbuf, rows, zbuf, acc_s, gsem, ssem):
        cid = lax.axis_index("c")
        sid = lax.axis_index("s")
        wid = sid * NC + cid
        pltpu.sync_copy(zeros_hbm.at[pl.ds(sid * NPS, NPS)], zbuf)
        pltpu.sync_copy(zbuf, acc_s.at[pl.ds(sid * NPS, NPS)])
        plsc.subcore_barrier()

        def window(w, carry):
            pltpu.sync_copy(src_hbm.at[w], srcbuf)
            pltpu.sync_copy(dst_hbm.at[w], dstbuf)
            gh = [
                pltpu.async_copy(g_hbm.at[srcbuf.at[j]], rows.at[j], gsem)
                for j in range(K)
            ]
            for h in gh:
                h.wait()
            sh = [
                pltpu.async_copy(rows.at[j], acc_s.at[dstbuf.at[j]], ssem,
                                 add=True)
                for j in range(K)
            ]
            for h in sh:
                h.wait()
            return carry

        lax.fori_loop(wid * WPT, (wid + 1) * WPT, window, 0)
        plsc.subcore_barrier()
        pltpu.sync_copy(acc_s.at[pl.ds(sid * NPS, NPS)], zbuf)
        pltpu.sync_copy(zbuf, out_hbm.at[pl.ds(cid * NP + sid * NPS, NPS)])

    return agg_kernel


def _make_dense1():
    @functools.partial(
        pl.kernel,
        out_type=(jax.ShapeDtypeStruct((NP,), _f32),
                  jax.ShapeDtypeStruct((CW * NP,), _f32)),
        mesh=_mesh,
        compiler_params=_params,
        scratch_types=[
            pltpu.VMEM((NPW,), _f32),
            pltpu.VMEM((NPW,), _f32),
            pltpu.VMEM((NPW,), _f32),
            pltpu.VMEM((2 * NPW,), _f32),
            pltpu.VMEM((CW * NPW,), _f32),
        ],
    )
    def dense1(degp_hbm, xf_hbm, dis_hbm, x2f_hbm, d0, d1, disb, xb, x2b):
        wid = lax.axis_index("s") * NC + lax.axis_index("c")
        r0 = wid * NPW
        pltpu.sync_copy(degp_hbm.at[pl.ds(r0, NPW)], d0)
        pltpu.sync_copy(degp_hbm.at[pl.ds(NP + r0, NPW)], d1)
        pltpu.sync_copy(xf_hbm.at[pl.ds(2 * r0, 2 * NPW)], xb)
        it = _iota()
        c8 = it & 7
        roff = it >> 3
        cmask = jnp.where(c8 < 2, 1.0, 0.0).astype(_f32)
        cclmp = jnp.minimum(c8, 1)

        def dis_step(i, carry):
            o = i * LANES
            d = d0[pl.ds(o, LANES)] + d1[pl.ds(o, LANES)] + 1.0
            disb[pl.ds(o, LANES)] = _rsqrt_newton(d)
            return carry

        lax.fori_loop(0, NPW // LANES, dis_step, 0)

        def x2_step(i, carry):
            o = i * LANES
            r = (o >> 3) + roff
            xv = plsc.load_gather(xb, [2 * r + cclmp]) * cmask
            dv = plsc.load_gather(disb, [r])
            x2b[pl.ds(o, LANES)] = dv * xv
            return carry

        lax.fori_loop(0, CW * NPW // LANES, x2_step, 0)
        pltpu.sync_copy(disb, dis_hbm.at[pl.ds(r0, NPW)])
        pltpu.sync_copy(x2b, x2f_hbm.at[pl.ds(CW * r0, CW * NPW)])

    return dense1


def _make_dense2():
    @functools.partial(
        pl.kernel,
        out_type=jax.ShapeDtypeStruct((CW * NP,), _f32),
        mesh=_mesh,
        compiler_params=_params,
        scratch_types=[
            pltpu.VMEM((CW * NPW,), _f32),
            pltpu.VMEM((CW * NPW,), _f32),
            pltpu.VMEM((NPW,), _f32),
            pltpu.VMEM((CW * NPW,), _f32),
            pltpu.VMEM((16,), _f32),
            pltpu.VMEM((16,), _f32),
        ],
    )
    def dense2(a0_hbm, a1_hbm, x2f_hbm, dis_hbm, w1_hbm, b1_hbm, g2f_hbm,
               aggb, tmpb, disb, g2b, wv, bv):
        wid = lax.axis_index("s") * NC + lax.axis_index("c")
        r0 = wid * NPW
        pltpu.sync_copy(a0_hbm.at[pl.ds(CW * r0, CW * NPW)], aggb)
        pltpu.sync_copy(a1_hbm.at[pl.ds(CW * r0, CW * NPW)], tmpb)
        pltpu.sync_copy(dis_hbm.at[pl.ds(r0, NPW)], disb)
        pltpu.sync_copy(w1_hbm, wv)
        pltpu.sync_copy(b1_hbm, bv)
        it = _iota()
        c8 = it & 7
        roff = it >> 3
        cmask = jnp.where(c8 < 4, 1.0, 0.0).astype(_f32)
        c4 = c8 & 3
        wv0 = plsc.load_gather(wv, [c4]) * cmask
        wv1 = plsc.load_gather(wv, [c4 + 4]) * cmask
        bvv = plsc.load_gather(bv, [c4]) * cmask

        def add_step(i, carry):
            o = i * LANES
            aggb[pl.ds(o, LANES)] = (aggb[pl.ds(o, LANES)]
                                     + tmpb[pl.ds(o, LANES)])
            return carry

        lax.fori_loop(0, CW * NPW // LANES, add_step, 0)
        pltpu.sync_copy(x2f_hbm.at[pl.ds(CW * r0, CW * NPW)], tmpb)

        def add2_step(i, carry):
            o = i * LANES
            aggb[pl.ds(o, LANES)] = (aggb[pl.ds(o, LANES)]
                                     + tmpb[pl.ds(o, LANES)])
            return carry

        lax.fori_loop(0, CW * NPW // LANES, add2_step, 0)

        def out_step(i, carry):
            o = i * LANES
            ib = o + CW * roff
            a0 = plsc.load_gather(aggb, [ib])
            a1 = plsc.load_gather(aggb, [ib + 1])
            dv = plsc.load_gather(disb, [(o >> 3) + roff])
            h = dv * (a0 * wv0 + a1 * wv1) + bvv
            h = jnp.maximum(h, 0.0)
            g2b[pl.ds(o, LANES)] = dv * h
            return carry

        lax.fori_loop(0, CW * NPW // LANES, out_step, 0)
        pltpu.sync_copy(g2b, g2f_hbm.at[pl.ds(CW * r0, CW * NPW)])

    return dense2


def _make_dense3():
    @functools.partial(
        pl.kernel,
        out_type=jax.ShapeDtypeStruct((2 * NP,), _f32),
        mesh=_mesh,
        compiler_params=_params,
        scratch_types=[
            pltpu.VMEM((CW * NPW,), _f32),
            pltpu.VMEM((CW * NPW,), _f32),
            pltpu.VMEM((CW * NPW,), _f32),
            pltpu.VMEM((NPW,), _f32),
            pltpu.VMEM((2 * NPW,), _f32),
            pltpu.VMEM((16,), _f32),
            pltpu.VMEM((16,), _f32),
            pltpu.VMEM((16,), _f32),
            pltpu.VMEM((16,), _f32),
        ],
    )
    def dense3(b0_hbm, b1_hbm, g2f_hbm, dis_hbm, w2_hbm, bb2_hbm, wl_hbm,
               bl_hbm, out_hbm, preb, tmpb, zb, disb, ob, w2v, b2v, wlv, blv):
        wid = lax.axis_index("s") * NC + lax.axis_index("c")
        r0 = wid * NPW
        pltpu.sync_copy(b0_hbm.at[pl.ds(CW * r0, CW * NPW)], preb)
        pltpu.sync_copy(b1_hbm.at[pl.ds(CW * r0, CW * NPW)], tmpb)
        pltpu.sync_copy(dis_hbm.at[pl.ds(r0, NPW)], disb)
        pltpu.sync_copy(w2_hbm, w2v)
        pltpu.sync_copy(bb2_hbm, b2v)
        pltpu.sync_copy(wl_hbm, wlv)
        pltpu.sync_copy(bl_hbm, blv)
        it = _iota()
        c8 = it & 7
        roff = it >> 3
        cmask = jnp.where(c8 < 4, 1.0, 0.0).astype(_f32)
        c4 = c8 & 3
        w2vs = [plsc.load_gather(w2v, [c4 + 4 * k]) * cmask for k in range(4)]
        b2vv = plsc.load_gather(b2v, [c4]) * cmask

        def add_step(i, carry):
            o = i * LANES
            preb[pl.ds(o, LANES)] = (preb[pl.ds(o, LANES)]
                                     + tmpb[pl.ds(o, LANES)])
            return carry

        lax.fori_loop(0, CW * NPW // LANES, add_step, 0)
        pltpu.sync_copy(g2f_hbm.at[pl.ds(CW * r0, CW * NPW)], tmpb)

        def pre_step(i, carry):
            o = i * LANES
            dv = plsc.load_gather(disb, [(o >> 3) + roff])
            preb[pl.ds(o, LANES)] = dv * (preb[pl.ds(o, LANES)]
                                          + tmpb[pl.ds(o, LANES)])
            return carry

        lax.fori_loop(0, CW * NPW // LANES, pre_step, 0)

        def z_step(i, carry):
            o = i * LANES
            ib = o + CW * roff
            acc = b2vv
            for k2 in range(4):
                pk = plsc.load_gather(preb, [ib + k2])
                acc = acc + pk * w2vs[k2]
            zb[pl.ds(o, LANES)] = jnp.maximum(acc, 0.0)
            return carry

        lax.fori_loop(0, CW * NPW // LANES, z_step, 0)

        lc2 = it & 1
        lr2 = it >> 1
        wlvs = [plsc.load_gather(wlv, [lc2 + 2 * j]) for j in range(4)]
        blvv = plsc.load_gather(blv, [lc2])

        def o_step(i, carry):
            o = i * LANES
            iz = 4 * o + CW * lr2
            acc = blvv
            for j in range(4):
                zj = plsc.load_gather(zb, [iz + j])
                acc = acc + zj * wlvs[j]
            ob[pl.ds(o, LANES)] = acc
            return carry

        lax.fori_loop(0, 2 * NPW // LANES, o_step, 0)
        pltpu.sync_copy(ob, out_hbm.at[pl.ds(2 * r0, 2 * NPW)])

    return dense3


_deg_k = _make_deg()
_agg_k = _make_agg()
_dense1_k = _make_dense1()
_dense2_k = _make_dense2()
_dense3_k = _make_dense3()


def _pad16(a):
    a = a.reshape(-1).astype(_f32)
    return jnp.pad(a, (0, 16 - a.shape[0]))


def kernel(x, edge_index, W1, b1, W2, b2, Wl, bl):
    ei = edge_index.astype(_i32)
    npad = EP - E
    pad_idx = N + (jnp.arange(npad, dtype=_i32) % (NP - N))
    src = jnp.concatenate([ei[0], pad_idx]).reshape(NWIN, K, 128)
    dst = jnp.concatenate([ei[1], pad_idx]).reshape(NWIN, K, 128)

    xf = jnp.pad(x.astype(_f32), ((0, NP - N), (0, 0))).reshape(-1)
    z1 = jnp.zeros((NP,), _f32)
    z8 = jnp.zeros((NP, CW), _f32)
    w1p, b1p = _pad16(W1), _pad16(b1)
    w2p, b2p = _pad16(W2), _pad16(b2)
    wlp, blp = _pad16(Wl), _pad16(bl)

    degp = _deg_k(dst, z1)
    dis, x2f = _dense1_k(degp, xf)
    acc1 = _agg_k(x2f.reshape(NP, CW), src, dst, z8)
    a1f = acc1.reshape(-1)
    g2f = _dense2_k(a1f[:CW * NP], a1f[CW * NP:], x2f, dis, w1p, b1p)
    acc2 = _agg_k(g2f.reshape(NP, CW), src, dst, z8)
    a2f = acc2.reshape(-1)
    outf = _dense3_k(a2f[:CW * NP], a2f[CW * NP:], g2f, dis,
                     w2p, b2p, wlp, blp)
    return outf.reshape(NP, 2)[:N]


# speedup vs baseline: 111.4157x; 1.4085x over previous
import functools

import jax
import jax.numpy as jnp
from jax import lax
from jax.experimental import pallas as pl
from jax.experimental.pallas import tpu as pltpu
from jax.experimental.pallas import tpu_sc as plsc

N = 100000
E = 6400000
NC, NS, LANES = 2, 16, 16
NW = NC * NS
NP = 100352
NPW = NP // NW
NPS = NP // NS
CW = 8
K = 8
WIN = K * 128
EP = 6422528
NWIN = EP // WIN
WPT = NWIN // NW

_mesh = plsc.VectorSubcoreMesh(core_axis_name="c", subcore_axis_name="s")
_params = pltpu.CompilerParams(needs_layout_passes=False,
                               use_tc_tiling_on_sc=False)
_f32 = jnp.float32
_i32 = jnp.int32


def _iota():
    return lax.iota(_i32, LANES)


def _rsqrt_newton(d):
    i = lax.bitcast_convert_type(d, _i32)
    y = lax.bitcast_convert_type(jnp.int32(0x5F3759DF) - (i >> 1), _f32)
    for _ in range(3):
        y = y * (1.5 - 0.5 * d * y * y)
    return y


PAIRS = (WPT - 2) // 2


def _make_deg():
    @functools.partial(
        pl.kernel,
        out_type=jax.ShapeDtypeStruct((NC * NP,), _f32),
        mesh=_mesh,
        compiler_params=_params,
        scratch_types=[
            pltpu.VMEM((K, 128), _i32),
            pltpu.VMEM((K, 128), _i32),
            pltpu.VMEM((128,), _f32),
            pltpu.VMEM((NPS,), _f32),
            pltpu.VMEM_SHARED((NP,), _f32),
            pltpu.SemaphoreType.DMA,
            pltpu.SemaphoreType.DMA,
        ],
    )
    def deg_kernel(dst_hbm, zeros_hbm, out_hbm, db0, db1, ones_v, zbuf,
                   deg_s, isem, ssem):
        cid = lax.axis_index("c")
        sid = lax.axis_index("s")
        wid = sid * NC + cid
        w0 = wid * WPT
        db = [db0, db1]
        pltpu.sync_copy(zeros_hbm.at[pl.ds(sid * NPS, NPS)], zbuf)
        pltpu.sync_copy(zbuf, deg_s.at[pl.ds(sid * NPS, NPS)])
        for i in range(128 // LANES):
            ones_v[pl.ds(i * LANES, LANES)] = jnp.full((LANES,), 1.0, _f32)
        plsc.subcore_barrier()

        def idx_async(w, buf):
            wc = jnp.minimum(w, NWIN - 1)
            pltpu.async_copy(dst_hbm.at[pl.ds(wc * K, K)], buf, isem)

        def idx_wait(buf):
            pltpu.make_async_copy(dst_hbm.at[pl.ds(0, K)], buf, isem).wait()

        def fire_s(buf):
            for j in range(K):
                pltpu.async_copy(ones_v, deg_s.at[buf.at[j]], ssem, add=True)

        def wait_s(buf):
            for j in range(K):
                pltpu.make_async_copy(ones_v, deg_s.at[buf.at[j]],
                                      ssem).wait()

        def half(pc, pn, w):
            idx_wait(db[pn])
            fire_s(db[pn])
            wait_s(db[pc])
            idx_async(w + 2, db[pc])

        pltpu.sync_copy(dst_hbm.at[pl.ds(w0 * K, K)], db0)
        fire_s(db0)
        idx_async(w0 + 1, db1)

        def body(i, carry):
            w = w0 + 2 * i
            half(0, 1, w)
            half(1, 0, w + 1)
            return carry

        lax.fori_loop(0, PAIRS, body, 0)
        half(0, 1, w0 + WPT - 2)
        wait_s(db[1])
        idx_wait(db[0])
        plsc.subcore_barrier()
        pltpu.sync_copy(deg_s.at[pl.ds(sid * NPS, NPS)], zbuf)
        pltpu.sync_copy(zbuf, out_hbm.at[pl.ds(cid * NP + sid * NPS, NPS)])

    return deg_kernel


def _make_agg():
    @functools.partial(
        pl.kernel,
        out_type=jax.ShapeDtypeStruct((NC * NP, CW), _f32),
        mesh=_mesh,
        compiler_params=_params,
        scratch_types=[
            pltpu.VMEM((K, 128), _i32),
            pltpu.VMEM((K, 128), _i32),
            pltpu.VMEM((K, 128), _i32),
            pltpu.VMEM((K, 128), _i32),
            pltpu.VMEM((K, 128, CW), _f32),
            pltpu.VMEM((K, 128, CW), _f32),
            pltpu.VMEM((NPS, CW), _f32),
            pltpu.VMEM_SHARED((NP, CW), _f32),
            pltpu.SemaphoreType.DMA,
            pltpu.SemaphoreType.DMA,
            pltpu.SemaphoreType.DMA,
        ],
    )
    def agg_kernel(g_hbm, src_hbm, dst_hbm, zeros_hbm, out_hbm,
                   sb0, sb1, db0, db1, rw0, rw1, zbuf, acc_s,
                   isem, gsem, ssem):
        cid = lax.axis_index("c")
        sid = lax.axis_index("s")
        wid = sid * NC + cid
        w0 = wid * WPT
        sb, db, rw = [sb0, sb1], [db0, db1], [rw0, rw1]
        pltpu.sync_copy(zeros_hbm.at[pl.ds(sid * NPS, NPS)], zbuf)
        pltpu.sync_copy(zbuf, acc_s.at[pl.ds(sid * NPS, NPS)])
        plsc.subcore_barrier()

        def idx_async(w, p):
            wc = jnp.minimum(w, NWIN - 1)
            pltpu.async_copy(src_hbm.at[pl.ds(wc * K, K)], sb[p], isem)
            pltpu.async_copy(dst_hbm.at[pl.ds(wc * K, K)], db[p], isem)

        def idx_wait(p):
            pltpu.make_async_copy(src_hbm.at[pl.ds(0, K)], sb[p], isem).wait()
            pltpu.make_async_copy(dst_hbm.at[pl.ds(0, K)], db[p], isem).wait()

        def fire_g(p):
            for j in range(K):
                pltpu.async_copy(g_hbm.at[sb[p].at[j]], rw[p].at[j], gsem)

        def wait_g(p):
            for j in range(K):
                pltpu.make_async_copy(g_hbm.at[sb[p].at[j]], rw[p].at[j],
                                      gsem).wait()

        def fire_s(p):
            for j in range(K):
                pltpu.async_copy(rw[p].at[j], acc_s.at[db[p].at[j]], ssem,
                                 add=True)

        def wait_s(p):
            for j in range(K):
                pltpu.make_async_copy(rw[p].at[j], acc_s.at[db[p].at[j]],
                                      ssem).wait()

        def half(pc, pn, w):
            idx_wait(pn)
            fire_g(pn)
            wait_g(pc)
            fire_s(pc)
            wait_s(pc)
            idx_async(w + 2, pc)

        pltpu.sync_copy(src_hbm.at[pl.ds(w0 * K, K)], sb0)
        pltpu.sync_copy(dst_hbm.at[pl.ds(w0 * K, K)], db0)
        fire_g(0)
        idx_async(w0 + 1, 1)

        def body(i, carry):
            w = w0 + 2 * i
            half(0, 1, w)
            half(1, 0, w + 1)
            return carry

        lax.fori_loop(0, PAIRS, body, 0)
        half(0, 1, w0 + WPT - 2)
        wait_g(1)
        fire_s(1)
        wait_s(1)
        idx_wait(0)
        plsc.subcore_barrier()
        pltpu.sync_copy(acc_s.at[pl.ds(sid * NPS, NPS)], zbuf)
        pltpu.sync_copy(zbuf, out_hbm.at[pl.ds(cid * NP + sid * NPS, NPS)])

    return agg_kernel


def _make_dense1():
    @functools.partial(
        pl.kernel,
        out_type=(jax.ShapeDtypeStruct((NP,), _f32),
                  jax.ShapeDtypeStruct((CW * NP,), _f32)),
        mesh=_mesh,
        compiler_params=_params,
        scratch_types=[
            pltpu.VMEM((NPW,), _f32),
            pltpu.VMEM((NPW,), _f32),
            pltpu.VMEM((NPW,), _f32),
            pltpu.VMEM((2 * NPW,), _f32),
            pltpu.VMEM((CW * NPW,), _f32),
        ],
    )
    def dense1(degp_hbm, xf_hbm, dis_hbm, x2f_hbm, d0, d1, disb, xb, x2b):
        wid = lax.axis_index("s") * NC + lax.axis_index("c")
        r0 = wid * NPW
        pltpu.sync_copy(degp_hbm.at[pl.ds(r0, NPW)], d0)
        pltpu.sync_copy(degp_hbm.at[pl.ds(NP + r0, NPW)], d1)
        pltpu.sync_copy(xf_hbm.at[pl.ds(2 * r0, 2 * NPW)], xb)
        it = _iota()
        c8 = it & 7
        roff = it >> 3
        cmask = jnp.where(c8 < 2, 1.0, 0.0).astype(_f32)
        cclmp = jnp.minimum(c8, 1)

        def dis_step(i, carry):
            o = i * LANES
            d = d0[pl.ds(o, LANES)] + d1[pl.ds(o, LANES)] + 1.0
            disb[pl.ds(o, LANES)] = _rsqrt_newton(d)
            return carry

        lax.fori_loop(0, NPW // LANES, dis_step, 0)

        def x2_step(i, carry):
            o = i * LANES
            r = (o >> 3) + roff
            xv = plsc.load_gather(xb, [2 * r + cclmp]) * cmask
            dv = plsc.load_gather(disb, [r])
            x2b[pl.ds(o, LANES)] = dv * xv
            return carry

        lax.fori_loop(0, CW * NPW // LANES, x2_step, 0)
        pltpu.sync_copy(disb, dis_hbm.at[pl.ds(r0, NPW)])
        pltpu.sync_copy(x2b, x2f_hbm.at[pl.ds(CW * r0, CW * NPW)])

    return dense1


def _make_dense2():
    @functools.partial(
        pl.kernel,
        out_type=jax.ShapeDtypeStruct((CW * NP,), _f32),
        mesh=_mesh,
        compiler_params=_params,
        scratch_types=[
            pltpu.VMEM((CW * NPW,), _f32),
            pltpu.VMEM((CW * NPW,), _f32),
            pltpu.VMEM((NPW,), _f32),
            pltpu.VMEM((CW * NPW,), _f32),
            pltpu.VMEM((16,), _f32),
            pltpu.VMEM((16,), _f32),
        ],
    )
    def dense2(a0_hbm, a1_hbm, x2f_hbm, dis_hbm, w1_hbm, b1_hbm, g2f_hbm,
               aggb, tmpb, disb, g2b, wv, bv):
        wid = lax.axis_index("s") * NC + lax.axis_index("c")
        r0 = wid * NPW
        pltpu.sync_copy(a0_hbm.at[pl.ds(CW * r0, CW * NPW)], aggb)
        pltpu.sync_copy(a1_hbm.at[pl.ds(CW * r0, CW * NPW)], tmpb)
        pltpu.sync_copy(dis_hbm.at[pl.ds(r0, NPW)], disb)
        pltpu.sync_copy(w1_hbm, wv)
        pltpu.sync_copy(b1_hbm, bv)
        it = _iota()
        c8 = it & 7
        roff = it >> 3
        cmask = jnp.where(c8 < 4, 1.0, 0.0).astype(_f32)
        c4 = c8 & 3
        wv0 = plsc.load_gather(wv, [c4]) * cmask
        wv1 = plsc.load_gather(wv, [c4 + 4]) * cmask
        bvv = plsc.load_gather(bv, [c4]) * cmask

        def add_step(i, carry):
            o = i * LANES
            aggb[pl.ds(o, LANES)] = (aggb[pl.ds(o, LANES)]
                                     + tmpb[pl.ds(o, LANES)])
            return carry

        lax.fori_loop(0, CW * NPW // LANES, add_step, 0)
        pltpu.sync_copy(x2f_hbm.at[pl.ds(CW * r0, CW * NPW)], tmpb)

        def add2_step(i, carry):
            o = i * LANES
            aggb[pl.ds(o, LANES)] = (aggb[pl.ds(o, LANES)]
                                     + tmpb[pl.ds(o, LANES)])
            return carry

        lax.fori_loop(0, CW * NPW // LANES, add2_step, 0)

        def out_step(i, carry):
            o = i * LANES
            ib = o + CW * roff
            a0 = plsc.load_gather(aggb, [ib])
            a1 = plsc.load_gather(aggb, [ib + 1])
            dv = plsc.load_gather(disb, [(o >> 3) + roff])
            h = dv * (a0 * wv0 + a1 * wv1) + bvv
            h = jnp.maximum(h, 0.0)
            g2b[pl.ds(o, LANES)] = dv * h
            return carry

        lax.fori_loop(0, CW * NPW // LANES, out_step, 0)
        pltpu.sync_copy(g2b, g2f_hbm.at[pl.ds(CW * r0, CW * NPW)])

    return dense2


def _make_dense3():
    @functools.partial(
        pl.kernel,
        out_type=jax.ShapeDtypeStruct((2 * NP,), _f32),
        mesh=_mesh,
        compiler_params=_params,
        scratch_types=[
            pltpu.VMEM((CW * NPW,), _f32),
            pltpu.VMEM((CW * NPW,), _f32),
            pltpu.VMEM((CW * NPW,), _f32),
            pltpu.VMEM((NPW,), _f32),
            pltpu.VMEM((2 * NPW,), _f32),
            pltpu.VMEM((16,), _f32),
            pltpu.VMEM((16,), _f32),
            pltpu.VMEM((16,), _f32),
            pltpu.VMEM((16,), _f32),
        ],
    )
    def dense3(b0_hbm, b1_hbm, g2f_hbm, dis_hbm, w2_hbm, bb2_hbm, wl_hbm,
               bl_hbm, out_hbm, preb, tmpb, zb, disb, ob, w2v, b2v, wlv, blv):
        wid = lax.axis_index("s") * NC + lax.axis_index("c")
        r0 = wid * NPW
        pltpu.sync_copy(b0_hbm.at[pl.ds(CW * r0, CW * NPW)], preb)
        pltpu.sync_copy(b1_hbm.at[pl.ds(CW * r0, CW * NPW)], tmpb)
        pltpu.sync_copy(dis_hbm.at[pl.ds(r0, NPW)], disb)
        pltpu.sync_copy(w2_hbm, w2v)
        pltpu.sync_copy(bb2_hbm, b2v)
        pltpu.sync_copy(wl_hbm, wlv)
        pltpu.sync_copy(bl_hbm, blv)
        it = _iota()
        c8 = it & 7
        roff = it >> 3
        cmask = jnp.where(c8 < 4, 1.0, 0.0).astype(_f32)
        c4 = c8 & 3
        w2vs = [plsc.load_gather(w2v, [c4 + 4 * k]) * cmask for k in range(4)]
        b2vv = plsc.load_gather(b2v, [c4]) * cmask

        def add_step(i, carry):
            o = i * LANES
            preb[pl.ds(o, LANES)] = (preb[pl.ds(o, LANES)]
                                     + tmpb[pl.ds(o, LANES)])
            return carry

        lax.fori_loop(0, CW * NPW // LANES, add_step, 0)
        pltpu.sync_copy(g2f_hbm.at[pl.ds(CW * r0, CW * NPW)], tmpb)

        def pre_step(i, carry):
            o = i * LANES
            dv = plsc.load_gather(disb, [(o >> 3) + roff])
            preb[pl.ds(o, LANES)] = dv * (preb[pl.ds(o, LANES)]
                                          + tmpb[pl.ds(o, LANES)])
            return carry

        lax.fori_loop(0, CW * NPW // LANES, pre_step, 0)

        def z_step(i, carry):
            o = i * LANES
            ib = o + CW * roff
            acc = b2vv
            for k2 in range(4):
                pk = plsc.load_gather(preb, [ib + k2])
                acc = acc + pk * w2vs[k2]
            zb[pl.ds(o, LANES)] = jnp.maximum(acc, 0.0)
            return carry

        lax.fori_loop(0, CW * NPW // LANES, z_step, 0)

        lc2 = it & 1
        lr2 = it >> 1
        wlvs = [plsc.load_gather(wlv, [lc2 + 2 * j]) for j in range(4)]
        blvv = plsc.load_gather(blv, [lc2])

        def o_step(i, carry):
            o = i * LANES
            iz = 4 * o + CW * lr2
            acc = blvv
            for j in range(4):
                zj = plsc.load_gather(zb, [iz + j])
                acc = acc + zj * wlvs[j]
            ob[pl.ds(o, LANES)] = acc
            return carry

        lax.fori_loop(0, 2 * NPW // LANES, o_step, 0)
        pltpu.sync_copy(ob, out_hbm.at[pl.ds(2 * r0, 2 * NPW)])

    return dense3


_deg_k = _make_deg()
_agg_k = _make_agg()
_dense1_k = _make_dense1()
_dense2_k = _make_dense2()
_dense3_k = _make_dense3()


def _pad16(a):
    a = a.reshape(-1).astype(_f32)
    return jnp.pad(a, (0, 16 - a.shape[0]))


def kernel(x, edge_index, W1, b1, W2, b2, Wl, bl):
    ei = edge_index.astype(_i32)
    npad = EP - E
    pad_idx = N + (jnp.arange(npad, dtype=_i32) % (NP - N))
    src = jnp.concatenate([ei[0], pad_idx]).reshape(NWIN * K, 128)
    dst = jnp.concatenate([ei[1], pad_idx]).reshape(NWIN * K, 128)

    xf = jnp.pad(x.astype(_f32), ((0, NP - N), (0, 0))).reshape(-1)
    z1 = jnp.zeros((NP,), _f32)
    z8 = jnp.zeros((NP, CW), _f32)
    w1p, b1p = _pad16(W1), _pad16(b1)
    w2p, b2p = _pad16(W2), _pad16(b2)
    wlp, blp = _pad16(Wl), _pad16(bl)

    degp = _deg_k(dst, z1)
    dis, x2f = _dense1_k(degp, xf)
    acc1 = _agg_k(x2f.reshape(NP, CW), src, dst, z8)
    a1f = acc1.reshape(-1)
    g2f = _dense2_k(a1f[:CW * NP], a1f[CW * NP:], x2f, dis, w1p, b1p)
    acc2 = _agg_k(g2f.reshape(NP, CW), src, dst, z8)
    a2f = acc2.reshape(-1)
    outf = _dense3_k(a2f[:CW * NP], a2f[CW * NP:], g2f, dis,
                     w2p, b2p, wlp, blp)
    return outf.reshape(NP, 2)[:N]


# speedup vs baseline: 116.3028x; 1.0439x over previous
import functools

import jax
import jax.numpy as jnp
from jax import lax
from jax.experimental import pallas as pl
from jax.experimental.pallas import tpu as pltpu
from jax.experimental.pallas import tpu_sc as plsc

N = 100000
E = 6400000
NC, NS, LANES = 2, 16, 16
NW = NC * NS
NP = 100352
NPW = NP // NW
NPS = NP // NS
CW = 8
K = 8
WIN = K * 128
EP = 6422528
NWIN = EP // WIN
WPT = NWIN // NW

_mesh = plsc.VectorSubcoreMesh(core_axis_name="c", subcore_axis_name="s")
_params = pltpu.CompilerParams(needs_layout_passes=False,
                               use_tc_tiling_on_sc=False)
_f32 = jnp.float32
_i32 = jnp.int32


def _iota():
    return lax.iota(_i32, LANES)


def _rsqrt_newton(d):
    i = lax.bitcast_convert_type(d, _i32)
    y = lax.bitcast_convert_type(jnp.int32(0x5F3759DF) - (i >> 1), _f32)
    for _ in range(3):
        y = y * (1.5 - 0.5 * d * y * y)
    return y


PAIRS = (WPT - 2) // 2
PAIRS_A = (WPT - 4) // 2


def _make_deg():
    @functools.partial(
        pl.kernel,
        out_type=jax.ShapeDtypeStruct((NC * NP,), _f32),
        mesh=_mesh,
        compiler_params=_params,
        scratch_types=[
            pltpu.VMEM((K, 128), _i32),
            pltpu.VMEM((K, 128), _i32),
            pltpu.VMEM((128,), _f32),
            pltpu.VMEM((NPS,), _f32),
            pltpu.VMEM_SHARED((NP,), _f32),
            pltpu.SemaphoreType.DMA,
            pltpu.SemaphoreType.DMA,
        ],
    )
    def deg_kernel(dst_hbm, zeros_hbm, out_hbm, db0, db1, ones_v, zbuf,
                   deg_s, isem, ssem):
        cid = lax.axis_index("c")
        sid = lax.axis_index("s")
        wid = sid * NC + cid
        w0 = wid * WPT
        db = [db0, db1]
        pltpu.sync_copy(zeros_hbm.at[pl.ds(sid * NPS, NPS)], zbuf)
        pltpu.sync_copy(zbuf, deg_s.at[pl.ds(sid * NPS, NPS)])
        for i in range(128 // LANES):
            ones_v[pl.ds(i * LANES, LANES)] = jnp.full((LANES,), 1.0, _f32)
        plsc.subcore_barrier()

        def idx_async(w, buf):
            wc = jnp.minimum(w, NWIN - 1)
            pltpu.async_copy(dst_hbm.at[pl.ds(wc * K, K)], buf, isem)

        def idx_wait(buf):
            pltpu.make_async_copy(dst_hbm.at[pl.ds(0, K)], buf, isem).wait()

        def fire_s(buf):
            for j in range(K):
                pltpu.async_copy(ones_v, deg_s.at[buf.at[j]], ssem, add=True)

        def wait_s(buf):
            for j in range(K):
                pltpu.make_async_copy(ones_v, deg_s.at[buf.at[j]],
                                      ssem).wait()

        def half(pc, pn, w):
            idx_wait(db[pn])
            fire_s(db[pn])
            wait_s(db[pc])
            idx_async(w + 2, db[pc])

        pltpu.sync_copy(dst_hbm.at[pl.ds(w0 * K, K)], db0)
        fire_s(db0)
        idx_async(w0 + 1, db1)

        def body(i, carry):
            w = w0 + 2 * i
            half(0, 1, w)
            half(1, 0, w + 1)
            return carry

        lax.fori_loop(0, PAIRS, body, 0)
        half(0, 1, w0 + WPT - 2)
        wait_s(db[1])
        idx_wait(db[0])
        plsc.subcore_barrier()
        pltpu.sync_copy(deg_s.at[pl.ds(sid * NPS, NPS)], zbuf)
        pltpu.sync_copy(zbuf, out_hbm.at[pl.ds(cid * NP + sid * NPS, NPS)])

    return deg_kernel


def _make_agg():
    @functools.partial(
        pl.kernel,
        out_type=jax.ShapeDtypeStruct((NC * NP, CW), _f32),
        mesh=_mesh,
        compiler_params=_params,
        scratch_types=[
            pltpu.VMEM((K, 128), _i32),
            pltpu.VMEM((K, 128), _i32),
            pltpu.VMEM((K, 128), _i32),
            pltpu.VMEM((K, 128), _i32),
            pltpu.VMEM((K, 128, CW), _f32),
            pltpu.VMEM((K, 128, CW), _f32),
            pltpu.VMEM((NPS, CW), _f32),
            pltpu.VMEM_SHARED((NP, CW), _f32),
            pltpu.SemaphoreType.DMA,
            pltpu.SemaphoreType.DMA,
            pltpu.SemaphoreType.DMA,
        ],
    )
    def agg_kernel(g_hbm, src_hbm, dst_hbm, zeros_hbm, out_hbm,
                   sb0, sb1, db0, db1, rw0, rw1, zbuf, acc_s,
                   isem, gsem, ssem):
        cid = lax.axis_index("c")
        sid = lax.axis_index("s")
        wid = sid * NC + cid
        w0 = wid * WPT
        sb, db, rw = [sb0, sb1], [db0, db1], [rw0, rw1]
        pltpu.sync_copy(zeros_hbm.at[pl.ds(sid * NPS, NPS)], zbuf)
        pltpu.sync_copy(zbuf, acc_s.at[pl.ds(sid * NPS, NPS)])
        plsc.subcore_barrier()

        def idx_async(w, p):
            wc = jnp.minimum(w, NWIN - 1)
            pltpu.async_copy(src_hbm.at[pl.ds(wc * K, K)], sb[p], isem)
            pltpu.async_copy(dst_hbm.at[pl.ds(wc * K, K)], db[p], isem)

        def idx_wait(p):
            pltpu.make_async_copy(src_hbm.at[pl.ds(0, K)], sb[p], isem).wait()
            pltpu.make_async_copy(dst_hbm.at[pl.ds(0, K)], db[p], isem).wait()

        def fire_g(p):
            for j in range(K):
                pltpu.async_copy(g_hbm.at[sb[p].at[j]], rw[p].at[j], gsem)

        def wait_g(p):
            for j in range(K):
                pltpu.make_async_copy(g_hbm.at[sb[p].at[j]], rw[p].at[j],
                                      gsem).wait()

        def fire_s(p):
            for j in range(K):
                pltpu.async_copy(rw[p].at[j], acc_s.at[db[p].at[j]], ssem,
                                 add=True)

        def wait_s(p):
            for j in range(K):
                pltpu.make_async_copy(rw[p].at[j], acc_s.at[db[p].at[j]],
                                      ssem).wait()

        def half(pc, pn, w):
            idx_wait(pn)
            wait_s(pn)
            fire_g(pn)
            wait_g(pc)
            fire_s(pc)
            idx_async(w + 2, pc)

        pltpu.sync_copy(src_hbm.at[pl.ds(w0 * K, K)], sb0)
        pltpu.sync_copy(dst_hbm.at[pl.ds(w0 * K, K)], db0)
        fire_g(0)
        idx_async(w0 + 1, 1)
        idx_wait(1)
        fire_g(1)
        wait_g(0)
        fire_s(0)
        idx_async(w0 + 2, 0)

        def body(i, carry):
            w = w0 + 1 + 2 * i
            half(1, 0, w)
            half(0, 1, w + 1)
            return carry

        lax.fori_loop(0, PAIRS_A, body, 0)
        half(1, 0, w0 + WPT - 3)
        half(0, 1, w0 + WPT - 2)
        wait_g(1)
        fire_s(1)
        wait_s(0)
        wait_s(1)
        idx_wait(0)
        plsc.subcore_barrier()
        pltpu.sync_copy(acc_s.at[pl.ds(sid * NPS, NPS)], zbuf)
        pltpu.sync_copy(zbuf, out_hbm.at[pl.ds(cid * NP + sid * NPS, NPS)])

    return agg_kernel


def _make_dense1():
    @functools.partial(
        pl.kernel,
        out_type=(jax.ShapeDtypeStruct((NP,), _f32),
                  jax.ShapeDtypeStruct((CW * NP,), _f32)),
        mesh=_mesh,
        compiler_params=_params,
        scratch_types=[
            pltpu.VMEM((NPW,), _f32),
            pltpu.VMEM((NPW,), _f32),
            pltpu.VMEM((NPW,), _f32),
            pltpu.VMEM((2 * NPW,), _f32),
            pltpu.VMEM((CW * NPW,), _f32),
        ],
    )
    def dense1(degp_hbm, xf_hbm, dis_hbm, x2f_hbm, d0, d1, disb, xb, x2b):
        wid = lax.axis_index("s") * NC + lax.axis_index("c")
        r0 = wid * NPW
        pltpu.sync_copy(degp_hbm.at[pl.ds(r0, NPW)], d0)
        pltpu.sync_copy(degp_hbm.at[pl.ds(NP + r0, NPW)], d1)
        pltpu.sync_copy(xf_hbm.at[pl.ds(2 * r0, 2 * NPW)], xb)
        it = _iota()
        c8 = it & 7
        roff = it >> 3
        cmask = jnp.where(c8 < 2, 1.0, 0.0).astype(_f32)
        cclmp = jnp.minimum(c8, 1)

        def dis_step(i, carry):
            o = i * LANES
            d = d0[pl.ds(o, LANES)] + d1[pl.ds(o, LANES)] + 1.0
            disb[pl.ds(o, LANES)] = _rsqrt_newton(d)
            return carry

        lax.fori_loop(0, NPW // LANES, dis_step, 0)

        def x2_step(i, carry):
            o = i * LANES
            r = (o >> 3) + roff
            xv = plsc.load_gather(xb, [2 * r + cclmp]) * cmask
            dv = plsc.load_gather(disb, [r])
            x2b[pl.ds(o, LANES)] = dv * xv
            return carry

        lax.fori_loop(0, CW * NPW // LANES, x2_step, 0)
        pltpu.sync_copy(disb, dis_hbm.at[pl.ds(r0, NPW)])
        pltpu.sync_copy(x2b, x2f_hbm.at[pl.ds(CW * r0, CW * NPW)])

    return dense1


def _make_dense2():
    @functools.partial(
        pl.kernel,
        out_type=jax.ShapeDtypeStruct((CW * NP,), _f32),
        mesh=_mesh,
        compiler_params=_params,
        scratch_types=[
            pltpu.VMEM((CW * NPW,), _f32),
            pltpu.VMEM((CW * NPW,), _f32),
            pltpu.VMEM((NPW,), _f32),
            pltpu.VMEM((CW * NPW,), _f32),
            pltpu.VMEM((16,), _f32),
            pltpu.VMEM((16,), _f32),
        ],
    )
    def dense2(a0_hbm, a1_hbm, x2f_hbm, dis_hbm, w1_hbm, b1_hbm, g2f_hbm,
               aggb, tmpb, disb, g2b, wv, bv):
        wid = lax.axis_index("s") * NC + lax.axis_index("c")
        r0 = wid * NPW
        pltpu.sync_copy(a0_hbm.at[pl.ds(CW * r0, CW * NPW)], aggb)
        pltpu.sync_copy(a1_hbm.at[pl.ds(CW * r0, CW * NPW)], tmpb)
        pltpu.sync_copy(dis_hbm.at[pl.ds(r0, NPW)], disb)
        pltpu.sync_copy(w1_hbm, wv)
        pltpu.sync_copy(b1_hbm, bv)
        it = _iota()
        c8 = it & 7
        roff = it >> 3
        cmask = jnp.where(c8 < 4, 1.0, 0.0).astype(_f32)
        c4 = c8 & 3
        wv0 = plsc.load_gather(wv, [c4]) * cmask
        wv1 = plsc.load_gather(wv, [c4 + 4]) * cmask
        bvv = plsc.load_gather(bv, [c4]) * cmask

        def add_step(i, carry):
            o = i * LANES
            aggb[pl.ds(o, LANES)] = (aggb[pl.ds(o, LANES)]
                                     + tmpb[pl.ds(o, LANES)])
            return carry

        lax.fori_loop(0, CW * NPW // LANES, add_step, 0)
        pltpu.sync_copy(x2f_hbm.at[pl.ds(CW * r0, CW * NPW)], tmpb)

        def add2_step(i, carry):
            o = i * LANES
            aggb[pl.ds(o, LANES)] = (aggb[pl.ds(o, LANES)]
                                     + tmpb[pl.ds(o, LANES)])
            return carry

        lax.fori_loop(0, CW * NPW // LANES, add2_step, 0)

        def out_step(i, carry):
            o = i * LANES
            ib = o + CW * roff
            a0 = plsc.load_gather(aggb, [ib])
            a1 = plsc.load_gather(aggb, [ib + 1])
            dv = plsc.load_gather(disb, [(o >> 3) + roff])
            h = dv * (a0 * wv0 + a1 * wv1) + bvv
            h = jnp.maximum(h, 0.0)
            g2b[pl.ds(o, LANES)] = dv * h
            return carry

        lax.fori_loop(0, CW * NPW // LANES, out_step, 0)
        pltpu.sync_copy(g2b, g2f_hbm.at[pl.ds(CW * r0, CW * NPW)])

    return dense2


def _make_dense3():
    @functools.partial(
        pl.kernel,
        out_type=jax.ShapeDtypeStruct((2 * NP,), _f32),
        mesh=_mesh,
        compiler_params=_params,
        scratch_types=[
            pltpu.VMEM((CW * NPW,), _f32),
            pltpu.VMEM((CW * NPW,), _f32),
            pltpu.VMEM((CW * NPW,), _f32),
            pltpu.VMEM((NPW,), _f32),
            pltpu.VMEM((2 * NPW,), _f32),
            pltpu.VMEM((16,), _f32),
            pltpu.VMEM((16,), _f32),
            pltpu.VMEM((16,), _f32),
            pltpu.VMEM((16,), _f32),
        ],
    )
    def dense3(b0_hbm, b1_hbm, g2f_hbm, dis_hbm, w2_hbm, bb2_hbm, wl_hbm,
               bl_hbm, out_hbm, preb, tmpb, zb, disb, ob, w2v, b2v, wlv, blv):
        wid = lax.axis_index("s") * NC + lax.axis_index("c")
        r0 = wid * NPW
        pltpu.sync_copy(b0_hbm.at[pl.ds(CW * r0, CW * NPW)], preb)
        pltpu.sync_copy(b1_hbm.at[pl.ds(CW * r0, CW * NPW)], tmpb)
        pltpu.sync_copy(dis_hbm.at[pl.ds(r0, NPW)], disb)
        pltpu.sync_copy(w2_hbm, w2v)
        pltpu.sync_copy(bb2_hbm, b2v)
        pltpu.sync_copy(wl_hbm, wlv)
        pltpu.sync_copy(bl_hbm, blv)
        it = _iota()
        c8 = it & 7
        roff = it >> 3
        cmask = jnp.where(c8 < 4, 1.0, 0.0).astype(_f32)
        c4 = c8 & 3
        w2vs = [plsc.load_gather(w2v, [c4 + 4 * k]) * cmask for k in range(4)]
        b2vv = plsc.load_gather(b2v, [c4]) * cmask

        def add_step(i, carry):
            o = i * LANES
            preb[pl.ds(o, LANES)] = (preb[pl.ds(o, LANES)]
                                     + tmpb[pl.ds(o, LANES)])
            return carry

        lax.fori_loop(0, CW * NPW // LANES, add_step, 0)
        pltpu.sync_copy(g2f_hbm.at[pl.ds(CW * r0, CW * NPW)], tmpb)

        def pre_step(i, carry):
            o = i * LANES
            dv = plsc.load_gather(disb, [(o >> 3) + roff])
            preb[pl.ds(o, LANES)] = dv * (preb[pl.ds(o, LANES)]
                                          + tmpb[pl.ds(o, LANES)])
            return carry

        lax.fori_loop(0, CW * NPW // LANES, pre_step, 0)

        def z_step(i, carry):
            o = i * LANES
            ib = o + CW * roff
            acc = b2vv
            for k2 in range(4):
                pk = plsc.load_gather(preb, [ib + k2])
                acc = acc + pk * w2vs[k2]
            zb[pl.ds(o, LANES)] = jnp.maximum(acc, 0.0)
            return carry

        lax.fori_loop(0, CW * NPW // LANES, z_step, 0)

        lc2 = it & 1
        lr2 = it >> 1
        wlvs = [plsc.load_gather(wlv, [lc2 + 2 * j]) for j in range(4)]
        blvv = plsc.load_gather(blv, [lc2])

        def o_step(i, carry):
            o = i * LANES
            iz = 4 * o + CW * lr2
            acc = blvv
            for j in range(4):
                zj = plsc.load_gather(zb, [iz + j])
                acc = acc + zj * wlvs[j]
            ob[pl.ds(o, LANES)] = acc
            return carry

        lax.fori_loop(0, 2 * NPW // LANES, o_step, 0)
        pltpu.sync_copy(ob, out_hbm.at[pl.ds(2 * r0, 2 * NPW)])

    return dense3


_deg_k = _make_deg()
_agg_k = _make_agg()
_dense1_k = _make_dense1()
_dense2_k = _make_dense2()
_dense3_k = _make_dense3()


def _pad16(a):
    a = a.reshape(-1).astype(_f32)
    return jnp.pad(a, (0, 16 - a.shape[0]))


def kernel(x, edge_index, W1, b1, W2, b2, Wl, bl):
    ei = edge_index.astype(_i32)
    npad = EP - E
    pad_idx = N + (jnp.arange(npad, dtype=_i32) % (NP - N))
    src = jnp.concatenate([ei[0], pad_idx]).reshape(NWIN * K, 128)
    dst = jnp.concatenate([ei[1], pad_idx]).reshape(NWIN * K, 128)

    xf = jnp.pad(x.astype(_f32), ((0, NP - N), (0, 0))).reshape(-1)
    z1 = jnp.zeros((NP,), _f32)
    z8 = jnp.zeros((NP, CW), _f32)
    w1p, b1p = _pad16(W1), _pad16(b1)
    w2p, b2p = _pad16(W2), _pad16(b2)
    wlp, blp = _pad16(Wl), _pad16(bl)

    degp = _deg_k(dst, z1)
    dis, x2f = _dense1_k(degp, xf)
    acc1 = _agg_k(x2f.reshape(NP, CW), src, dst, z8)
    a1f = acc1.reshape(-1)
    g2f = _dense2_k(a1f[:CW * NP], a1f[CW * NP:], x2f, dis, w1p, b1p)
    acc2 = _agg_k(g2f.reshape(NP, CW), src, dst, z8)
    a2f = acc2.reshape(-1)
    outf = _dense3_k(a2f[:CW * NP], a2f[CW * NP:], g2f, dis,
                     w2p, b2p, wlp, blp)
    return outf.reshape(NP, 2)[:N]
